# baseline (speedup 1.0000x reference)
import jax
import jax.numpy as jnp
from jax.experimental import pallas as pl
from jax.experimental.pallas import tpu as pltpu


def _mlp_kernel(x_ref, w1_ref, b1_ref, w2_ref, b2_ref, o_ref,
                w1s_ref, w2s_ref):
    @pl.when(pl.program_id(1) == 0)
    def _cast_weights():
        w1s_ref[...] = w1_ref[...].astype(jnp.bfloat16)
        w2s_ref[...] = w2_ref[...].astype(jnp.bfloat16)

    h = jax.lax.dot_general(
        x_ref[...].astype(jnp.bfloat16), w1s_ref[...],
        (((1,), (1,)), ((), ())),
        preferred_element_type=jnp.float32)
    h = (h + b1_ref[...]).astype(jnp.bfloat16)
    o = jax.lax.dot_general(
        h, w2s_ref[...], (((1,), (1,)), ((), ())),
        preferred_element_type=jnp.float32)
    o_ref[...] = (o + b2_ref[...]).astype(o_ref.dtype)


def _pick_tile(n, candidates):
    for c in candidates:
        if n % c == 0:
            return c
    return n


def kernel(x, w1, b1, w2, b2):
    B, D_in = x.shape
    H = w1.shape[0]
    D_out = w2.shape[0]

    b1r = b1.reshape(1, H)
    b2r = b2.reshape(1, D_out)

    if B % 2 == 0:
        cores, tb = 2, _pick_tile(B // 2, (1024, 512, 256, 128, 8))
        inner = B // 2 // tb
    else:
        cores, tb = 1, _pick_tile(B, (1024, 512, 256, 128, 8))
        inner = B // tb

    out = pl.pallas_call(
        _mlp_kernel,
        grid=(cores, inner),
        in_specs=[
            pl.BlockSpec((tb, D_in), lambda c, j, n=inner: (c * n + j, 0)),
            pl.BlockSpec((H, D_in), lambda c, j: (0, 0)),
            pl.BlockSpec((1, H), lambda c, j: (0, 0)),
            pl.BlockSpec((D_out, H), lambda c, j: (0, 0)),
            pl.BlockSpec((1, D_out), lambda c, j: (0, 0)),
        ],
        out_specs=pl.BlockSpec((tb, D_out), lambda c, j, n=inner: (c * n + j, 0)),
        out_shape=jax.ShapeDtypeStruct((B, D_out), x.dtype),
        scratch_shapes=[
            pltpu.VMEM((H, D_in), jnp.bfloat16),
            pltpu.VMEM((D_out, H), jnp.bfloat16),
        ],
        compiler_params=pltpu.CompilerParams(
            dimension_semantics=("parallel", "arbitrary")),
    )(x, w1, b1r, w2, b2r)
    return out


# speedup vs baseline: 1.2372x; 1.2372x over previous
import jax
import jax.numpy as jnp
from jax.experimental import pallas as pl
from jax.experimental.pallas import tpu as pltpu


def _mlp_kernel(x_ref, w1_ref, b1_ref, w2_ref, b2_ref, o_ref,
                wts_ref, bias_ref):
    @pl.when(pl.program_id(1) == 0)
    def _fuse():
        wt = jax.lax.dot_general(
            w1_ref[...].astype(jnp.bfloat16),
            w2_ref[...].astype(jnp.bfloat16),
            (((0,), (1,)), ((), ())),
            preferred_element_type=jnp.float32)
        wts_ref[...] = wt.astype(jnp.bfloat16)
        bias_ref[...] = b2_ref[...] + jax.lax.dot_general(
            b1_ref[...], w2_ref[...], (((1,), (1,)), ((), ())),
            preferred_element_type=jnp.float32)

    acc = jnp.dot(x_ref[...].astype(jnp.bfloat16), wts_ref[...],
                  preferred_element_type=jnp.float32)
    o_ref[...] = (acc + bias_ref[...]).astype(o_ref.dtype)


def _pick_tile(n, candidates):
    for c in candidates:
        if n % c == 0:
            return c
    return n


def kernel(x, w1, b1, w2, b2):
    B, D_in = x.shape
    H = w1.shape[0]
    D_out = w2.shape[0]

    b1r = b1.reshape(1, H)
    b2r = b2.reshape(1, D_out)

    if B % 2 == 0:
        cores, tb = 2, _pick_tile(B // 2, (1024, 512, 256, 128, 8))
        inner = B // 2 // tb
    else:
        cores, tb = 1, _pick_tile(B, (1024, 512, 256, 128, 8))
        inner = B // tb

    out = pl.pallas_call(
        _mlp_kernel,
        grid=(cores, inner),
        in_specs=[
            pl.BlockSpec((tb, D_in), lambda c, j, n=inner: (c * n + j, 0)),
            pl.BlockSpec((H, D_in), lambda c, j: (0, 0)),
            pl.BlockSpec((1, H), lambda c, j: (0, 0)),
            pl.BlockSpec((D_out, H), lambda c, j: (0, 0)),
            pl.BlockSpec((1, D_out), lambda c, j: (0, 0)),
        ],
        out_specs=pl.BlockSpec((tb, D_out), lambda c, j, n=inner: (c * n + j, 0)),
        out_shape=jax.ShapeDtypeStruct((B, D_out), x.dtype),
        scratch_shapes=[
            pltpu.VMEM((D_in, D_out), jnp.bfloat16),
            pltpu.VMEM((1, D_out), jnp.float32),
        ],
        compiler_params=pltpu.CompilerParams(
            dimension_semantics=("parallel", "arbitrary")),
    )(x, w1, b1r, w2, b2r)
    return out


# speedup vs baseline: 1.2451x; 1.0064x over previous
import jax
import jax.numpy as jnp
from jax.experimental import pallas as pl
from jax.experimental.pallas import tpu as pltpu


def _fuse_kernel(w1_ref, w2_ref, b1_ref, b2_ref, wt_ref, b_ref):
    wt = jax.lax.dot_general(
        w1_ref[...].astype(jnp.bfloat16),
        w2_ref[...].astype(jnp.bfloat16),
        (((0,), (1,)), ((), ())),
        preferred_element_type=jnp.float32)
    wt_ref[...] = wt.astype(jnp.bfloat16)
    b_ref[...] = b2_ref[...] + jax.lax.dot_general(
        b1_ref[...], w2_ref[...], (((1,), (1,)), ((), ())),
        preferred_element_type=jnp.float32)


def _mlp_kernel(x_ref, wt_ref, b_ref, o_ref):
    acc = jnp.dot(x_ref[...].astype(jnp.bfloat16), wt_ref[...],
                  preferred_element_type=jnp.float32)
    o_ref[...] = (acc + b_ref[...]).astype(o_ref.dtype)


def _pick_tile(n, candidates):
    for c in candidates:
        if n % c == 0:
            return c
    return n


def kernel(x, w1, b1, w2, b2):
    B, D_in = x.shape
    H = w1.shape[0]
    D_out = w2.shape[0]

    b1r = b1.reshape(1, H)
    b2r = b2.reshape(1, D_out)

    wt, bias = pl.pallas_call(
        _fuse_kernel,
        in_specs=[
            pl.BlockSpec(memory_space=pltpu.MemorySpace.VMEM),
            pl.BlockSpec(memory_space=pltpu.MemorySpace.VMEM),
            pl.BlockSpec(memory_space=pltpu.MemorySpace.VMEM),
            pl.BlockSpec(memory_space=pltpu.MemorySpace.VMEM),
        ],
        out_specs=[
            pl.BlockSpec(memory_space=pltpu.MemorySpace.VMEM),
            pl.BlockSpec(memory_space=pltpu.MemorySpace.VMEM),
        ],
        out_shape=[
            jax.ShapeDtypeStruct((D_in, D_out), jnp.bfloat16),
            jax.ShapeDtypeStruct((1, D_out), jnp.float32),
        ],
    )(w1, w2, b1r, b2r)

    tb = _pick_tile(B, (1024, 512, 256, 128, 8))
    out = pl.pallas_call(
        _mlp_kernel,
        grid=(B // tb,),
        in_specs=[
            pl.BlockSpec((tb, D_in), lambda i: (i, 0)),
            pl.BlockSpec((D_in, D_out), lambda i: (0, 0)),
            pl.BlockSpec((1, D_out), lambda i: (0, 0)),
        ],
        out_specs=pl.BlockSpec((tb, D_out), lambda i: (i, 0)),
        out_shape=jax.ShapeDtypeStruct((B, D_out), x.dtype),
        compiler_params=pltpu.CompilerParams(
            dimension_semantics=("parallel",)),
    )(x, wt, bias)
    return out
